# baseline (speedup 1.0000x reference)
import functools

import jax
import jax.numpy as jnp
from jax import lax
from jax.experimental import pallas as pl
from jax.experimental.pallas import tpu as pltpu
from jax.experimental.pallas import tpu_sc as plsc

_ALPHA = 0.1
_EPS_THRESH = 1e-05

_NS = 16
_L = 16
_EC = 128
_W = 16


def _make_pagerank(N, E):
  chunk = _NS * _L
  Np = ((N + chunk - 1) // chunk) * chunk
  if Np == N:
    Np += chunk
  C = Np // _NS
  CV = C // _L
  ept = (E + _NS - 1) // _NS
  NCH = (ept + _EC - 1) // _EC
  NCH = ((NCH + 7) // 8) * 8
  Etp = NCH * _EC
  Ep = _NS * Etp
  RV = _EC // _L

  mesh = plsc.VectorSubcoreMesh(
      core_axis_name="c", subcore_axis_name="s", num_cores=1, num_subcores=_NS
  )

  @functools.partial(
      pl.kernel,
      out_type=jax.ShapeDtypeStruct((Np,), jnp.float32),
      mesh=mesh,
      compiler_params=pltpu.CompilerParams(needs_layout_passes=False),
      scratch_types=[
          pltpu.VMEM((NCH, _EC), jnp.int32),
          pltpu.VMEM((NCH, _EC), jnp.int32),
          pltpu.VMEM((NCH, _EC), jnp.float32),
          pltpu.VMEM((Np,), jnp.float32),
          pltpu.VMEM((C,), jnp.float32),
          pltpu.VMEM((C,), jnp.float32),
          pltpu.VMEM((C,), jnp.float32),
          pltpu.VMEM((C,), jnp.float32),
          pltpu.VMEM((_L,), jnp.float32),
          pltpu.VMEM((_NS, _L), jnp.float32),
          pltpu.VMEM_SHARED((Np,), jnp.float32),
          pltpu.VMEM_SHARED((Np,), jnp.float32),
          pltpu.VMEM_SHARED((_NS, _L), jnp.float32),
          pltpu.SemaphoreType.DMA,
      ],
  )
  def pagerank(src_hbm, dst_hbm, pi0_hbm, out_hbm,
               src2, dst2, msg2, q_v, dis_v, pis_v, sbuf_v, zbuf_v, tmp_v,
               eps_v, acc_s, vec_s, eps_s, rsem):
    sid = lax.axis_index("s")
    nbase = sid * C

    pltpu.sync_copy(src_hbm.at[pl.ds(sid * NCH, NCH), :], src2)
    pltpu.sync_copy(dst_hbm.at[pl.ds(sid * NCH, NCH), :], dst2)
    pltpu.sync_copy(pi0_hbm.at[pl.ds(nbase, C)], pis_v)

    zeros = jnp.zeros((_L,), jnp.float32)
    ones = jnp.ones((_L,), jnp.float32)
    lane = lax.iota(jnp.int32, _L)

    def zbody(j, c):
      zbuf_v[pl.ds(j * _L, _L)] = zeros
      return c

    lax.fori_loop(0, CV, zbody, 0)

    def start_row_scatter(r):
      pltpu.async_copy(msg2.at[r], acc_s.at[dst2.at[r]], rsem, add=True)

    def wait_row_scatter():
      pltpu.make_async_copy(msg2.at[0], acc_s.at[dst2.at[0]], rsem).wait()

    def drain_tail():
      def wbody(j, c):
        wait_row_scatter()
        return c
      lax.fori_loop(0, _W, wbody, 0)

    def scatter_all(make_row):
      def rbody(r, c):
        make_row(r)
        start_row_scatter(r)

        @pl.when(r >= _W)
        def _():
          wait_row_scatter()
        return c

      lax.fori_loop(0, NCH, rbody, 0)
      drain_tail()

    pltpu.sync_copy(zbuf_v, acc_s.at[pl.ds(nbase, C)])

    def ones_row(r):
      for u in range(RV):
        msg2[r, pl.ds(u * _L, _L)] = ones

    plsc.subcore_barrier()
    scatter_all(ones_row)
    plsc.subcore_barrier()
    pltpu.sync_copy(acc_s.at[pl.ds(nbase, C)], sbuf_v)

    def dis_body(j, carry):
      deg = sbuf_v[pl.ds(j * _L, _L)] + 1.0
      i = plsc.bitcast(deg, jnp.int32)
      i = jnp.int32(0x5F3759DF) - lax.shift_right_logical(i, 1)
      y = plsc.bitcast(i, jnp.float32)
      for _ in range(3):
        y = y * (1.5 - 0.5 * deg * y * y)
      gi = jnp.int32(nbase + j * _L) + lane
      y = jnp.where(gi < N, y, 0.0)
      dis_v[pl.ds(j * _L, _L)] = y
      sbuf_v[pl.ds(j * _L, _L)] = y * pis_v[pl.ds(j * _L, _L)]
      return carry

    lax.fori_loop(0, CV, dis_body, 0)

    pltpu.sync_copy(sbuf_v, vec_s.at[pl.ds(nbase, C)])
    plsc.subcore_barrier()
    pltpu.sync_copy(vec_s, q_v)

    teleport = jnp.float32(_ALPHA / N)
    damp = jnp.float32(1.0 - _ALPHA)

    def it_body(carry):
      pltpu.sync_copy(zbuf_v, acc_s.at[pl.ds(nbase, C)])
      plsc.subcore_barrier()

      def gather_row(r):
        for u in range(RV):
          s = src2[r, pl.ds(u * _L, _L)]
          msg2[r, pl.ds(u * _L, _L)] = plsc.load_gather(q_v, [s])

      scatter_all(gather_row)
      plsc.subcore_barrier()
      pltpu.sync_copy(acc_s.at[pl.ds(nbase, C)], sbuf_v)

      def new_body(j, sq):
        tot = sbuf_v[pl.ds(j * _L, _L)]
        old = pis_v[pl.ds(j * _L, _L)]
        dis = dis_v[pl.ds(j * _L, _L)]
        gi = jnp.int32(nbase + j * _L) + lane
        pin = damp * dis * (tot + dis * old) + teleport
        pin = jnp.where(gi < N, pin, 0.0)
        pis_v[pl.ds(j * _L, _L)] = pin
        sbuf_v[pl.ds(j * _L, _L)] = dis * pin
        dlt = pin - old
        return sq + dlt * dlt

      sq = lax.fori_loop(0, CV, new_body, zeros)
      tmp_v[...] = sq
      pltpu.sync_copy(sbuf_v, vec_s.at[pl.ds(nbase, C)])
      pltpu.sync_copy(tmp_v, eps_s.at[sid])
      plsc.subcore_barrier()
      pltpu.sync_copy(vec_s, q_v)
      pltpu.sync_copy(eps_s, eps_v)
      tot16 = eps_v[0]
      for t in range(1, _NS):
        tot16 = tot16 + eps_v[t]
      return jnp.sum(tot16)

    thresh = jnp.float32(_EPS_THRESH) * jnp.float32(_EPS_THRESH)
    lax.while_loop(lambda s: s > thresh, it_body, jnp.float32(1e10))

    pltpu.sync_copy(pis_v, out_hbm.at[pl.ds(nbase, C)])

  return pagerank, Np, Ep


def kernel(x, edge_index):
  N = x.shape[0]
  E = edge_index.shape[1]
  fn, Np, Ep = _make_pagerank(N, E)

  src = edge_index[0]
  dst = edge_index[1]
  if Ep > E:
    pad = jnp.full((Ep - E,), N, dtype=jnp.int32)
    src = jnp.concatenate([src, pad])
    dst = jnp.concatenate([dst, pad])
  src = src.reshape(Ep // _EC, _EC)
  dst = dst.reshape(Ep // _EC, _EC)

  kpi = jax.random.key(42)
  pi0 = jax.random.uniform(kpi, (N, 1), dtype=jnp.float32)
  pi0 = pi0 / jnp.sum(jnp.abs(pi0))
  pi0 = jnp.pad(pi0[:, 0], (0, Np - N))

  out = fn(src, dst, pi0)
  return out[:N, None]


# speedup vs baseline: 1.1437x; 1.1437x over previous
import functools

import jax
import jax.numpy as jnp
from jax import lax
from jax.experimental import pallas as pl
from jax.experimental.pallas import tpu as pltpu
from jax.experimental.pallas import tpu_sc as plsc

_ALPHA = 0.1
_EPS_THRESH = 1e-05

_NS = 16
_L = 16


def _make_pagerank(N, E):
  chunk = _NS * _L
  Np = ((N + chunk - 1) // chunk) * chunk
  if Np == N:
    Np += chunk
  C = Np // _NS
  Ep = ((E + chunk - 1) // chunk) * chunk
  Et = Ep // _NS
  NV = Np // _L
  CV = C // _L
  EV = Et // _L
  UE = next(u for u in (50, 25, 10, 8, 5, 4, 2, 1) if EV % u == 0)
  UZ = next(u for u in (16, 8, 4, 2, 1) if NV % u == 0)

  mesh = plsc.VectorSubcoreMesh(
      core_axis_name="c", subcore_axis_name="s", num_cores=1, num_subcores=_NS
  )

  @functools.partial(
      pl.kernel,
      out_type=jax.ShapeDtypeStruct((Np,), jnp.float32),
      mesh=mesh,
      compiler_params=pltpu.CompilerParams(needs_layout_passes=False),
      scratch_types=[
          pltpu.VMEM((Et,), jnp.int32),
          pltpu.VMEM((Et,), jnp.int32),
          pltpu.VMEM((Np,), jnp.float32),
          pltpu.VMEM((Np,), jnp.float32),
          pltpu.VMEM((C,), jnp.float32),
          pltpu.VMEM((C,), jnp.float32),
          pltpu.VMEM((C,), jnp.float32),
          pltpu.VMEM((_NS, C), jnp.float32),
          pltpu.VMEM((_L,), jnp.float32),
          pltpu.VMEM((_NS, _L), jnp.float32),
          pltpu.VMEM_SHARED((_NS, Np), jnp.float32),
          pltpu.VMEM_SHARED((2, Np), jnp.float32),
          pltpu.VMEM_SHARED((Np,), jnp.float32),
          pltpu.VMEM_SHARED((2, _NS, _L), jnp.float32),
          pltpu.SemaphoreType.DMA,
          pltpu.SemaphoreType.DMA,
      ],
  )
  def pagerank(src_hbm, dst_hbm, pi0_hbm, out_hbm,
               src_v, dst_v, q_v, acc_v, dis_v, pis_v, qsl_v, red_v, tmp_v,
               eps_v, stage_s, vec_s, zero_s, eps_s, rsem, zsem):
    sid = lax.axis_index("s")
    ebase = sid * Et
    nbase = sid * C

    pltpu.sync_copy(src_hbm.at[pl.ds(ebase, Et)], src_v)
    pltpu.sync_copy(dst_hbm.at[pl.ds(ebase, Et)], dst_v)
    pltpu.sync_copy(pi0_hbm.at[pl.ds(nbase, C)], pis_v)

    zeros = jnp.zeros((_L,), jnp.float32)
    ones = jnp.ones((_L,), jnp.float32)
    lane = lax.iota(jnp.int32, _L)

    def zero_acc():
      def zbody(j, c):
        for u in range(UZ):
          acc_v[pl.ds((j * UZ + u) * _L, _L)] = zeros
        return c
      lax.fori_loop(0, NV // UZ, zbody, 0)

    def fetch_stage_rows():
      copies = [
          pltpu.make_async_copy(
              stage_s.at[t, pl.ds(nbase, C)], red_v.at[t], rsem)
          for t in range(_NS)
      ]
      for cp in copies:
        cp.start()
      for cp in copies:
        cp.wait()

    def reduce_slice(j, fn):
      tot = red_v[0, pl.ds(j * _L, _L)]
      for t in range(1, _NS):
        tot = tot + red_v[t, pl.ds(j * _L, _L)]
      return fn(j, tot)

    zero_acc()
    pltpu.sync_copy(acc_v.at[pl.ds(nbase, C)], zero_s.at[pl.ds(nbase, C)])

    def deg_body(j, c):
      for u in range(UE):
        d = dst_v[pl.ds((j * UE + u) * _L, _L)]
        plsc.addupdate_scatter(acc_v, [d], ones)
      return c

    lax.fori_loop(0, EV // UE, deg_body, 0)

    pltpu.sync_copy(acc_v, stage_s.at[sid])
    plsc.subcore_barrier()
    pltpu.async_copy(zero_s, acc_v, zsem)
    fetch_stage_rows()

    def dis_body(j, carry):
      def finish(j, deg):
        deg = deg + 1.0
        i = plsc.bitcast(deg, jnp.int32)
        i = jnp.int32(0x5F3759DF) - lax.shift_right_logical(i, 1)
        y = plsc.bitcast(i, jnp.float32)
        for _ in range(3):
          y = y * (1.5 - 0.5 * deg * y * y)
        gi = jnp.int32(nbase + j * _L) + lane
        y = jnp.where(gi < N, y, 0.0)
        dis_v[pl.ds(j * _L, _L)] = y
        qsl_v[pl.ds(j * _L, _L)] = y * pis_v[pl.ds(j * _L, _L)]
        return 0
      return reduce_slice(j, finish)

    lax.fori_loop(0, CV, dis_body, 0)

    pltpu.sync_copy(qsl_v, vec_s.at[0, pl.ds(nbase, C)])
    plsc.subcore_barrier()
    pltpu.sync_copy(vec_s.at[0], q_v)

    teleport = jnp.float32(_ALPHA / N)
    damp = jnp.float32(1.0 - _ALPHA)

    def it_body(carry):
      _, p = carry
      pnew = 1 - p
      pltpu.make_async_copy(zero_s, acc_v, zsem).wait()

      def edge_body(j, c):
        for u in range(UE):
          s = src_v[pl.ds((j * UE + u) * _L, _L)]
          d = dst_v[pl.ds((j * UE + u) * _L, _L)]
          plsc.addupdate_scatter(acc_v, [d], plsc.load_gather(q_v, [s]))
        return c

      lax.fori_loop(0, EV // UE, edge_body, 0)

      pltpu.sync_copy(acc_v, stage_s.at[sid])
      plsc.subcore_barrier()
      pltpu.async_copy(zero_s, acc_v, zsem)
      fetch_stage_rows()

      def new_body(j, sq):
        def finish(j, tot):
          old = pis_v[pl.ds(j * _L, _L)]
          dis = dis_v[pl.ds(j * _L, _L)]
          gi = jnp.int32(nbase + j * _L) + lane
          pin = damp * dis * (tot + dis * old) + teleport
          pin = jnp.where(gi < N, pin, 0.0)
          pis_v[pl.ds(j * _L, _L)] = pin
          qsl_v[pl.ds(j * _L, _L)] = dis * pin
          dlt = pin - old
          return sq + dlt * dlt
        return reduce_slice(j, finish)

      sq = lax.fori_loop(0, CV, new_body, zeros)
      tmp_v[...] = sq
      w1 = pltpu.make_async_copy(qsl_v, vec_s.at[pnew, pl.ds(nbase, C)], rsem)
      w2 = pltpu.make_async_copy(tmp_v, eps_s.at[pnew, sid], rsem)
      w1.start(); w2.start(); w1.wait(); w2.wait()
      plsc.subcore_barrier()
      r1 = pltpu.make_async_copy(vec_s.at[pnew], q_v, rsem)
      r2 = pltpu.make_async_copy(eps_s.at[pnew], eps_v, rsem)
      r1.start(); r2.start(); r1.wait(); r2.wait()
      tot16 = eps_v[0]
      for t in range(1, _NS):
        tot16 = tot16 + eps_v[t]
      return jnp.sum(tot16), pnew

    thresh = jnp.float32(_EPS_THRESH) * jnp.float32(_EPS_THRESH)
    lax.while_loop(lambda c: c[0] > thresh, it_body,
                   (jnp.float32(1e10), jnp.int32(0)))

    pltpu.make_async_copy(zero_s, acc_v, zsem).wait()
    pltpu.sync_copy(pis_v, out_hbm.at[pl.ds(nbase, C)])

  return pagerank, Np, Ep


def kernel(x, edge_index):
  N = x.shape[0]
  E = edge_index.shape[1]
  fn, Np, Ep = _make_pagerank(N, E)

  src = edge_index[0]
  dst = edge_index[1]
  if Ep > E:
    pad = jnp.full((Ep - E,), N, dtype=jnp.int32)
    src = jnp.concatenate([src, pad])
    dst = jnp.concatenate([dst, pad])

  kpi = jax.random.key(42)
  pi0 = jax.random.uniform(kpi, (N, 1), dtype=jnp.float32)
  pi0 = pi0 / jnp.sum(jnp.abs(pi0))
  pi0 = jnp.pad(pi0[:, 0], (0, Np - N))

  out = fn(src, dst, pi0)
  return out[:N, None]


# speedup vs baseline: 1.1542x; 1.0092x over previous
import functools

import jax
import jax.numpy as jnp
from jax import lax
from jax.experimental import pallas as pl
from jax.experimental.pallas import tpu as pltpu
from jax.experimental.pallas import tpu_sc as plsc

_ALPHA = 0.1
_EPS_THRESH = 1e-05

_NS = 16
_L = 16


def _make_pagerank(N, E):
  chunk = _NS * _L
  Np = ((N + chunk - 1) // chunk) * chunk
  if Np == N:
    Np += chunk
  C = Np // _NS
  Ep = ((E + chunk - 1) // chunk) * chunk
  Et = Ep // _NS
  NV = Np // _L
  CV = C // _L
  EV = Et // _L
  UE = next(u for u in (50, 25, 10, 8, 5, 4, 2, 1) if EV % u == 0)
  UZ = next(u for u in (16, 8, 4, 2, 1) if NV % u == 0)

  mesh = plsc.VectorSubcoreMesh(
      core_axis_name="c", subcore_axis_name="s", num_cores=1, num_subcores=_NS
  )

  @functools.partial(
      pl.kernel,
      out_type=jax.ShapeDtypeStruct((Np,), jnp.float32),
      mesh=mesh,
      compiler_params=pltpu.CompilerParams(needs_layout_passes=False),
      scratch_types=[
          pltpu.VMEM((Et,), jnp.int32),
          pltpu.VMEM((Et,), jnp.int32),
          pltpu.VMEM((Np,), jnp.float32),
          pltpu.VMEM((Np,), jnp.float32),
          pltpu.VMEM((C,), jnp.float32),
          pltpu.VMEM((C,), jnp.float32),
          pltpu.VMEM((C,), jnp.float32),
          pltpu.VMEM((_NS, C), jnp.float32),
          pltpu.VMEM((_L,), jnp.float32),
          pltpu.VMEM((_NS, _L), jnp.float32),
          pltpu.VMEM_SHARED((_NS, Np), jnp.float32),
          pltpu.VMEM_SHARED((2, Np), jnp.float32),
          pltpu.VMEM_SHARED((Np,), jnp.float32),
          pltpu.VMEM_SHARED((2, _NS, _L), jnp.float32),
          pltpu.SemaphoreType.DMA,
          pltpu.SemaphoreType.DMA,
      ],
  )
  def pagerank(src_hbm, dst_hbm, pi0_hbm, out_hbm,
               src_v, dst_v, q_v, acc_v, dis_v, pis_v, qsl_v, red_v, tmp_v,
               eps_v, stage_s, vec_s, zero_s, eps_s, rsem, zsem):
    sid = lax.axis_index("s")
    ebase = sid * Et
    nbase = sid * C

    loads = [
        pltpu.make_async_copy(src_hbm.at[pl.ds(ebase, Et)], src_v, rsem),
        pltpu.make_async_copy(dst_hbm.at[pl.ds(ebase, Et)], dst_v, rsem),
        pltpu.make_async_copy(pi0_hbm.at[pl.ds(nbase, C)], pis_v, rsem),
    ]
    for cp in loads:
      cp.start()
    for cp in loads:
      cp.wait()

    zeros = jnp.zeros((_L,), jnp.float32)
    ones = jnp.ones((_L,), jnp.float32)
    lane = lax.iota(jnp.int32, _L)

    def zero_acc():
      def zbody(j, c):
        for u in range(UZ):
          acc_v[pl.ds((j * UZ + u) * _L, _L)] = zeros
        return c
      lax.fori_loop(0, NV // UZ, zbody, 0)

    def fetch_stage_rows():
      copies = [
          pltpu.make_async_copy(
              stage_s.at[t, pl.ds(nbase, C)], red_v.at[t], rsem)
          for t in range(_NS)
      ]
      for cp in copies:
        cp.start()
      for cp in copies:
        cp.wait()

    def reduce_slice(j, fn):
      tot = red_v[0, pl.ds(j * _L, _L)]
      for t in range(1, _NS):
        tot = tot + red_v[t, pl.ds(j * _L, _L)]
      return fn(j, tot)

    zero_acc()
    pltpu.sync_copy(acc_v.at[pl.ds(nbase, C)], zero_s.at[pl.ds(nbase, C)])

    def deg_body(j, c):
      for u in range(UE):
        d = dst_v[pl.ds((j * UE + u) * _L, _L)]
        plsc.addupdate_scatter(acc_v, [d], ones)
      return c

    lax.fori_loop(0, EV // UE, deg_body, 0)

    pltpu.sync_copy(acc_v, stage_s.at[sid])
    plsc.subcore_barrier()
    pltpu.async_copy(zero_s, acc_v, zsem)
    fetch_stage_rows()

    def dis_body(j, carry):
      def finish(j, deg):
        deg = deg + 1.0
        i = plsc.bitcast(deg, jnp.int32)
        i = jnp.int32(0x5F3759DF) - lax.shift_right_logical(i, 1)
        y = plsc.bitcast(i, jnp.float32)
        for _ in range(3):
          y = y * (1.5 - 0.5 * deg * y * y)
        gi = jnp.int32(nbase + j * _L) + lane
        y = jnp.where(gi < N, y, 0.0)
        dis_v[pl.ds(j * _L, _L)] = y
        qsl_v[pl.ds(j * _L, _L)] = y * pis_v[pl.ds(j * _L, _L)]
        return 0
      return reduce_slice(j, finish)

    lax.fori_loop(0, CV, dis_body, 0)

    pltpu.sync_copy(qsl_v, vec_s.at[0, pl.ds(nbase, C)])
    plsc.subcore_barrier()
    pltpu.sync_copy(vec_s.at[0], q_v)

    teleport = jnp.float32(_ALPHA / N)
    damp = jnp.float32(1.0 - _ALPHA)

    def it_body(carry):
      _, p = carry
      pnew = 1 - p
      pltpu.make_async_copy(zero_s, acc_v, zsem).wait()

      def edge_body(j, c):
        for u in range(UE):
          s = src_v[pl.ds((j * UE + u) * _L, _L)]
          d = dst_v[pl.ds((j * UE + u) * _L, _L)]
          plsc.addupdate_scatter(acc_v, [d], plsc.load_gather(q_v, [s]))
        return c

      lax.fori_loop(0, EV // UE, edge_body, 0)

      pltpu.sync_copy(acc_v, stage_s.at[sid])
      plsc.subcore_barrier()
      pltpu.async_copy(zero_s, acc_v, zsem)
      fetch_stage_rows()

      def new_body(j, sq):
        def finish(j, tot):
          old = pis_v[pl.ds(j * _L, _L)]
          dis = dis_v[pl.ds(j * _L, _L)]
          gi = jnp.int32(nbase + j * _L) + lane
          pin = damp * dis * (tot + dis * old) + teleport
          pin = jnp.where(gi < N, pin, 0.0)
          pis_v[pl.ds(j * _L, _L)] = pin
          qsl_v[pl.ds(j * _L, _L)] = dis * pin
          dlt = pin - old
          return sq + dlt * dlt
        return reduce_slice(j, finish)

      sq = lax.fori_loop(0, CV, new_body, zeros)
      tmp_v[...] = sq
      w1 = pltpu.make_async_copy(qsl_v, vec_s.at[pnew, pl.ds(nbase, C)], rsem)
      w2 = pltpu.make_async_copy(tmp_v, eps_s.at[pnew, sid], rsem)
      w1.start(); w2.start(); w1.wait(); w2.wait()
      plsc.subcore_barrier()
      r1 = pltpu.make_async_copy(vec_s.at[pnew], q_v, rsem)
      r2 = pltpu.make_async_copy(eps_s.at[pnew], eps_v, rsem)
      r1.start(); r2.start(); r1.wait(); r2.wait()
      tot16 = eps_v[0]
      for t in range(1, _NS):
        tot16 = tot16 + eps_v[t]
      return jnp.sum(tot16), pnew

    thresh = jnp.float32(_EPS_THRESH) * jnp.float32(_EPS_THRESH)
    lax.while_loop(lambda c: c[0] > thresh, it_body,
                   (jnp.float32(1e10), jnp.int32(0)))

    pltpu.make_async_copy(zero_s, acc_v, zsem).wait()
    pltpu.sync_copy(pis_v, out_hbm.at[pl.ds(nbase, C)])

  return pagerank, Np, Ep


def kernel(x, edge_index):
  N = x.shape[0]
  E = edge_index.shape[1]
  fn, Np, Ep = _make_pagerank(N, E)

  src = edge_index[0]
  dst = edge_index[1]
  if Ep > E:
    pad = jnp.full((Ep - E,), N, dtype=jnp.int32)
    src = jnp.concatenate([src, pad])
    dst = jnp.concatenate([dst, pad])

  kpi = jax.random.key(42)
  pi0 = jax.random.uniform(kpi, (N, 1), dtype=jnp.float32)
  pi0 = pi0 / jnp.sum(jnp.abs(pi0))
  pi0 = jnp.pad(pi0[:, 0], (0, Np - N))

  out = fn(src, dst, pi0)
  return out[:N, None]
